# Initial kernel scaffold; baseline (speedup 1.0000x reference)
import jax
import jax.numpy as jnp
from jax.experimental import pallas as pl
from jax.experimental.pallas import tpu as pltpu
from jax.experimental.pallas import tpu_sc as plsc

_WINDOW = 128


def kernel(token_ids, weight):
    batch, seq = token_ids.shape
    num_indices = batch * seq
    dim = weight.shape[1]
    idx_flat = token_ids.reshape(1, num_indices).astype(jnp.int32)

    mesh = plsc.VectorSubcoreMesh(core_axis_name="core", subcore_axis_name="subcore")

    @pl.kernel(
        out_type=jax.ShapeDtypeStruct((num_indices, dim), weight.dtype),
        mesh=mesh,
    )
    def gather_kernel(w_hbm, i_hbm, o_hbm):
        def body(i_vmem, o_vmem):
            pltpu.sync_copy(w_hbm.at[i_vmem.at[0]], o_vmem)

        pltpu.emit_pipeline(
            body,
            grid=(num_indices // _WINDOW,),
            in_specs=[pl.BlockSpec((1, _WINDOW), index_map=lambda i: (0, i))],
            out_specs=[pl.BlockSpec((_WINDOW, dim), index_map=lambda i: (i, 0))],
            core_axis_name=("core", "subcore"),
            dimension_semantics=(pltpu.PARALLEL,),
        )(i_hbm, o_hbm)

    out = gather_kernel(weight, idx_flat)
    return out.reshape(batch, seq, dim)


# speedup vs baseline: 1.6820x; 1.6820x over previous
import jax
import jax.numpy as jnp
from jax import lax
from jax.experimental import pallas as pl
from jax.experimental.pallas import tpu as pltpu
from jax.experimental.pallas import tpu_sc as plsc

_NUM_CORES = 2
_NUM_SUBCORES = 16
_NUM_WORKERS = _NUM_CORES * _NUM_SUBCORES
_CHUNK = 128


def kernel(token_ids, weight):
    batch, seq = token_ids.shape
    num_indices = batch * seq
    dim = weight.shape[1]
    per_worker = num_indices // _NUM_WORKERS
    n_chunks = per_worker // _CHUNK
    idx_flat = token_ids.reshape(num_indices).astype(jnp.int32)

    mesh = plsc.VectorSubcoreMesh(core_axis_name="c", subcore_axis_name="s")

    @pl.kernel(
        out_type=jax.ShapeDtypeStruct((num_indices, dim), weight.dtype),
        mesh=mesh,
        compiler_params=pltpu.CompilerParams(use_tc_tiling_on_sc=False),
        scratch_types=[
            pltpu.VMEM((per_worker,), jnp.int32),
            pltpu.VMEM((_CHUNK, dim), jnp.float32),
            pltpu.SemaphoreType.DMA,
        ],
    )
    def gather_kernel(w_hbm, i_hbm, o_hbm, idx_v, rows_v, sem):
        wid = lax.axis_index("s") * _NUM_CORES + lax.axis_index("c")
        base = wid * per_worker
        pltpu.sync_copy(i_hbm.at[pl.ds(base, per_worker)], idx_v)

        @pl.loop(0, n_chunks)
        def _(c):
            off = c * _CHUNK
            pltpu.async_copy(
                w_hbm.at[idx_v.at[pl.ds(off, _CHUNK)]], rows_v, sem
            ).wait()
            pltpu.sync_copy(rows_v, o_hbm.at[pl.ds(base + off, _CHUNK)])

    out = gather_kernel(weight, idx_flat)
    return out.reshape(batch, seq, dim)


# speedup vs baseline: 1.8303x; 1.0882x over previous
import jax
import jax.numpy as jnp
from jax import lax
from jax.experimental import pallas as pl
from jax.experimental.pallas import tpu as pltpu
from jax.experimental.pallas import tpu_sc as plsc

_NUM_CORES = 2
_NUM_SUBCORES = 16
_NUM_WORKERS = _NUM_CORES * _NUM_SUBCORES
_CHUNK = 512


def kernel(token_ids, weight):
    batch, seq = token_ids.shape
    num_indices = batch * seq
    dim = weight.shape[1]
    per_worker = num_indices // _NUM_WORKERS
    n_chunks = per_worker // _CHUNK
    idx_flat = token_ids.reshape(num_indices).astype(jnp.int32)

    mesh = plsc.VectorSubcoreMesh(core_axis_name="c", subcore_axis_name="s")

    @pl.kernel(
        out_type=jax.ShapeDtypeStruct((num_indices, dim), weight.dtype),
        mesh=mesh,
        compiler_params=pltpu.CompilerParams(use_tc_tiling_on_sc=False),
        scratch_types=[
            pltpu.VMEM((per_worker,), jnp.int32),
            pltpu.VMEM((_CHUNK, dim), jnp.float32),
            pltpu.SemaphoreType.DMA,
        ],
    )
    def gather_kernel(w_hbm, i_hbm, o_hbm, idx_v, rows_v, sem):
        wid = lax.axis_index("s") * _NUM_CORES + lax.axis_index("c")
        base = wid * per_worker
        pltpu.sync_copy(i_hbm.at[pl.ds(base, per_worker)], idx_v)

        @pl.loop(0, n_chunks)
        def _(c):
            off = c * _CHUNK
            pltpu.async_copy(
                w_hbm.at[idx_v.at[pl.ds(off, _CHUNK)]], rows_v, sem
            ).wait()
            pltpu.sync_copy(rows_v, o_hbm.at[pl.ds(base + off, _CHUNK)])

    out = gather_kernel(weight, idx_flat)
    return out.reshape(batch, seq, dim)


# speedup vs baseline: 1.8745x; 1.0242x over previous
import jax
import jax.numpy as jnp
from jax import lax
from jax.experimental import pallas as pl
from jax.experimental.pallas import tpu as pltpu
from jax.experimental.pallas import tpu_sc as plsc

_NUM_CORES = 2
_NUM_SUBCORES = 16
_NUM_WORKERS = _NUM_CORES * _NUM_SUBCORES
_CHUNK = 200
_K = 4


def kernel(token_ids, weight):
    batch, seq = token_ids.shape
    num_indices = batch * seq
    dim = weight.shape[1]
    per_worker = num_indices // _NUM_WORKERS
    n_chunks = per_worker // _CHUNK
    n_groups = n_chunks // _K
    idx_flat = token_ids.reshape(num_indices).astype(jnp.int32)

    mesh = plsc.VectorSubcoreMesh(core_axis_name="c", subcore_axis_name="s")

    @pl.kernel(
        out_type=jax.ShapeDtypeStruct((num_indices, dim), weight.dtype),
        mesh=mesh,
        compiler_params=pltpu.CompilerParams(use_tc_tiling_on_sc=False),
        scratch_types=[
            pltpu.VMEM((per_worker,), jnp.int32),
            pltpu.VMEM((2, _K, _CHUNK, dim), jnp.float32),
            pltpu.SemaphoreType.DMA,
            pltpu.SemaphoreType.DMA,
            pltpu.SemaphoreType.DMA,
            pltpu.SemaphoreType.DMA,
        ],
    )
    def gather_kernel(w_hbm, i_hbm, o_hbm, idx_v, rows, gs_a, ss_a, gs_b, ss_b):
        wid = lax.axis_index("s") * _NUM_CORES + lax.axis_index("c")
        base = wid * per_worker
        pltpu.sync_copy(i_hbm.at[pl.ds(base, per_worker)], idx_v)

        def issue_gathers(set_i, group, gsem):
            for b in range(_K):
                off = (group * _K + b) * _CHUNK
                pltpu.async_copy(
                    w_hbm.at[idx_v.at[pl.ds(off, _CHUNK)]], rows.at[set_i, b], gsem
                )

        def drain_gathers(set_i, gsem):
            for b in range(_K):
                pltpu.make_async_copy(
                    o_hbm.at[pl.ds(0, _CHUNK)], rows.at[set_i, b], gsem
                ).wait()

        def issue_stores(set_i, group, ssem):
            for b in range(_K):
                off = (group * _K + b) * _CHUNK
                pltpu.async_copy(
                    rows.at[set_i, b], o_hbm.at[pl.ds(base + off, _CHUNK)], ssem
                )

        def drain_stores(set_i, group, ssem):
            for b in range(_K):
                off = (group * _K + b) * _CHUNK
                pltpu.make_async_copy(
                    rows.at[set_i, b], o_hbm.at[pl.ds(base + off, _CHUNK)], ssem
                ).wait()

        issue_gathers(0, 0, gs_a)
        issue_gathers(1, 1, gs_b)

        @pl.loop(0, n_groups, step=2)
        def _(g):
            drain_gathers(0, gs_a)
            issue_stores(0, g, ss_a)
            drain_stores(0, g, ss_a)

            @pl.when(g + 2 < n_groups)
            def _():
                issue_gathers(0, g + 2, gs_a)

            drain_gathers(1, gs_b)
            issue_stores(1, g + 1, ss_b)
            drain_stores(1, g + 1, ss_b)

            @pl.when(g + 3 < n_groups)
            def _():
                issue_gathers(1, g + 3, gs_b)

    out = gather_kernel(weight, idx_flat)
    return out.reshape(batch, seq, dim)
